# baseline (speedup 1.0000x reference)
import functools

import jax
import jax.numpy as jnp
from jax import lax
from jax.experimental import pallas as pl
from jax.experimental.pallas import tpu as pltpu
from jax.experimental.pallas import tpu_sc as plsc

NUM_ITEMS = 1000000
LATENT_DIM = 64
BATCH = 16384

NC = 2
NS = 16
L = 16
NW = NC * NS
BPW = BATCH // NW
CHUNK = 128
NCHUNK = BPW // CHUNK

BLKW = 32768
SCN = 32768
IB0 = SCN // BLKW
NBLK_TC = (NUM_ITEMS - SCN + BLKW - 1) // BLKW

WPW = SCN // NW
CW = 512
NCH = WPW // CW
GR = CW // L


def _matvec_body(w_ref, et_ref, b_ref, y_ref):
    y_ref[...] = jnp.dot(w_ref[...], et_ref[...],
                         preferred_element_type=jnp.float32) + b_ref[0, 0]


def _scmv_body(et_hbm, wt_hbm, b16_hbm, ysc_hbm,
               buf0, buf1, wt_v, b_v, y_v, sem0, sem1):
    wid = lax.axis_index("s") * NC + lax.axis_index("c")
    base = wid * WPW
    pltpu.sync_copy(wt_hbm, wt_v)
    pltpu.sync_copy(b16_hbm, b_v)
    pltpu.async_copy(et_hbm.at[:, pl.ds(base, CW)], buf0, sem0)

    def compute(buf, cbase):
        def fbody(f, accs):
            wf = wt_v[pl.ds(f * L, L)]
            return tuple(a + buf[f, pl.ds(g * L, L)] * wf
                         for g, a in enumerate(accs))
        accs = lax.fori_loop(0, LATENT_DIM, fbody,
                             tuple(b_v[...] for _ in range(GR)))
        for g in range(GR):
            y_v[pl.ds(cbase + g * L, L)] = accs[g]

    def pair(k, carry):
        c0 = 2 * k
        pltpu.make_async_copy(et_hbm.at[:, pl.ds(0, CW)], buf0, sem0).wait()
        pltpu.async_copy(
            et_hbm.at[:, pl.ds(base + (c0 + 1) * CW, CW)], buf1, sem1)
        compute(buf0, c0 * CW)
        pltpu.make_async_copy(et_hbm.at[:, pl.ds(0, CW)], buf1, sem1).wait()

        @pl.when(k + 1 < NCH // 2)
        def _():
            pltpu.async_copy(
                et_hbm.at[:, pl.ds(base + (c0 + 2) * CW, CW)], buf0, sem0)

        compute(buf1, (c0 + 1) * CW)
        return carry

    lax.fori_loop(0, NCH // 2, pair, 0)
    pltpu.sync_copy(y_v, ysc_hbm.at[pl.ds(base, WPW)])


def _gather_body(y_hbm, idx_hbm, out_hbm, idx_v, out_v, sem):
    wid = lax.axis_index("s") * NC + lax.axis_index("c")
    base = wid * BPW
    for c in range(NCHUNK):
        pltpu.sync_copy(idx_hbm.at[pl.ds(base + c * CHUNK, CHUNK)],
                        idx_v.at[c])
    copies = []
    for c in range(NCHUNK):
        copies.append(pltpu.async_copy(
            y_hbm.at[idx_v.at[c]],
            out_v.at[pl.ds(c * CHUNK, CHUNK)], sem))
    for cp in copies:
        cp.wait()
    pltpu.sync_copy(out_v, out_hbm.at[pl.ds(base, BPW)])


_SC_MESH = dict(core_axis_name="c", subcore_axis_name="s",
                num_cores=NC, num_subcores=NS)


@jax.jit
def kernel(item_indices, item_emb, ffn_w, ffn_b):
    idx = item_indices.astype(jnp.int32)
    et = jnp.swapaxes(item_emb, 0, 1)
    b2 = ffn_b.reshape(1, 1)
    w = ffn_w.reshape(LATENT_DIM).astype(jnp.float32)
    wt = jnp.repeat(w, L)
    b16 = jnp.broadcast_to(ffn_b.astype(jnp.float32), (L,))

    sc_matvec = pl.kernel(
        _scmv_body,
        out_type=jax.ShapeDtypeStruct((SCN,), jnp.float32),
        mesh=plsc.VectorSubcoreMesh(**_SC_MESH),
        compiler_params=pltpu.CompilerParams(needs_layout_passes=False),
        scratch_types=[
            pltpu.VMEM((LATENT_DIM, CW), jnp.float32),
            pltpu.VMEM((LATENT_DIM, CW), jnp.float32),
            pltpu.VMEM((LATENT_DIM * L,), jnp.float32),
            pltpu.VMEM((L,), jnp.float32),
            pltpu.VMEM((WPW,), jnp.float32),
            pltpu.SemaphoreType.DMA,
            pltpu.SemaphoreType.DMA,
        ],
    )
    y_sc = sc_matvec(et, wt, b16)

    y2 = pl.pallas_call(
        _matvec_body,
        grid=(NBLK_TC,),
        in_specs=[
            pl.BlockSpec((1, LATENT_DIM), lambda i: (0, 0)),
            pl.BlockSpec((LATENT_DIM, BLKW), lambda i: (0, i + IB0)),
            pl.BlockSpec((1, 1), lambda i: (0, 0), memory_space=pltpu.SMEM),
        ],
        out_specs=pl.BlockSpec((1, BLKW), lambda i: (0, i + IB0)),
        out_shape=jax.ShapeDtypeStruct((1, NUM_ITEMS), jnp.float32),
        compiler_params=pltpu.CompilerParams(
            dimension_semantics=("parallel",)),
    )(ffn_w, et, b2)
    y_tc = y2.reshape(NUM_ITEMS)
    y = lax.concatenate([y_sc, lax.slice(y_tc, (SCN,), (NUM_ITEMS,))], 0)

    gather = pl.kernel(
        _gather_body,
        out_type=jax.ShapeDtypeStruct((BATCH,), jnp.float32),
        mesh=plsc.VectorSubcoreMesh(**_SC_MESH),
        compiler_params=pltpu.CompilerParams(needs_layout_passes=False),
        scratch_types=[
            pltpu.VMEM((NCHUNK, CHUNK), jnp.int32),
            pltpu.VMEM((BPW,), jnp.float32),
            pltpu.SemaphoreType.DMA,
        ],
    )
    out = gather(y, idx)
    return out.reshape(BATCH, 1)


# speedup vs baseline: 1.1521x; 1.1521x over previous
import functools

import jax
import jax.numpy as jnp
from jax import lax
from jax.experimental import pallas as pl
from jax.experimental.pallas import tpu as pltpu
from jax.experimental.pallas import tpu_sc as plsc

NUM_ITEMS = 1000000
LATENT_DIM = 64
BATCH = 16384

NC = 2
NS = 16
L = 16
NW = NC * NS
BPW = BATCH // NW
CHUNK = 128
NCHUNK = BPW // CHUNK

BLKW = 32768
SCN = 688128
IB0 = SCN // BLKW
NBLK_TC = (NUM_ITEMS - SCN + BLKW - 1) // BLKW

WPW = SCN // NW
CW = 512
NCH = WPW // CW
GR = CW // L


def _matvec_body(w_ref, et_ref, b_ref, y_ref):
    y_ref[...] = jnp.dot(w_ref[...], et_ref[...],
                         preferred_element_type=jnp.float32) + b_ref[0, 0]


def _scmv_body(et_hbm, wt_hbm, b16_hbm, ysc_hbm,
               buf0, buf1, wt_v, b_v, yw0, yw1, sem0, sem1, osem0, osem1):
    wid = lax.axis_index("s") * NC + lax.axis_index("c")
    base = wid * WPW
    pltpu.sync_copy(wt_hbm, wt_v)
    pltpu.sync_copy(b16_hbm, b_v)
    pltpu.async_copy(et_hbm.at[:, pl.ds(base, CW)], buf0, sem0)

    def compute(buf, yw):
        def fbody(f, accs):
            wf = wt_v[pl.ds(f * L, L)]
            return tuple(a + buf[f, pl.ds(g * L, L)] * wf
                         for g, a in enumerate(accs))
        accs = lax.fori_loop(0, LATENT_DIM, fbody,
                             tuple(b_v[...] for _ in range(GR)))
        for g in range(GR):
            yw[pl.ds(g * L, L)] = accs[g]

    def pair(k, carry):
        c0 = 2 * k
        pltpu.make_async_copy(et_hbm.at[:, pl.ds(0, CW)], buf0, sem0).wait()
        pltpu.async_copy(
            et_hbm.at[:, pl.ds(base + (c0 + 1) * CW, CW)], buf1, sem1)

        @pl.when(k > 0)
        def _():
            pltpu.make_async_copy(
                yw0, ysc_hbm.at[pl.ds(0, CW)], osem0).wait()

        compute(buf0, yw0)
        pltpu.async_copy(yw0, ysc_hbm.at[pl.ds(base + c0 * CW, CW)], osem0)
        pltpu.make_async_copy(et_hbm.at[:, pl.ds(0, CW)], buf1, sem1).wait()

        @pl.when(k + 1 < NCH // 2)
        def _():
            pltpu.async_copy(
                et_hbm.at[:, pl.ds(base + (c0 + 2) * CW, CW)], buf0, sem0)

        @pl.when(k > 0)
        def _():
            pltpu.make_async_copy(
                yw1, ysc_hbm.at[pl.ds(0, CW)], osem1).wait()

        compute(buf1, yw1)
        pltpu.async_copy(
            yw1, ysc_hbm.at[pl.ds(base + (c0 + 1) * CW, CW)], osem1)
        return carry

    lax.fori_loop(0, NCH // 2, pair, 0)
    pltpu.make_async_copy(yw0, ysc_hbm.at[pl.ds(0, CW)], osem0).wait()
    pltpu.make_async_copy(yw1, ysc_hbm.at[pl.ds(0, CW)], osem1).wait()


def _gather_body(y_hbm, idx_hbm, out_hbm, idx_v, out_v, sem):
    wid = lax.axis_index("s") * NC + lax.axis_index("c")
    base = wid * BPW
    for c in range(NCHUNK):
        pltpu.sync_copy(idx_hbm.at[pl.ds(base + c * CHUNK, CHUNK)],
                        idx_v.at[c])
    copies = []
    for c in range(NCHUNK):
        copies.append(pltpu.async_copy(
            y_hbm.at[idx_v.at[c]],
            out_v.at[pl.ds(c * CHUNK, CHUNK)], sem))
    for cp in copies:
        cp.wait()
    pltpu.sync_copy(out_v, out_hbm.at[pl.ds(base, BPW)])


_SC_MESH = dict(core_axis_name="c", subcore_axis_name="s",
                num_cores=NC, num_subcores=NS)


@jax.jit
def kernel(item_indices, item_emb, ffn_w, ffn_b):
    idx = item_indices.astype(jnp.int32)
    et = jnp.swapaxes(item_emb, 0, 1)
    b2 = ffn_b.reshape(1, 1)
    w = ffn_w.reshape(LATENT_DIM).astype(jnp.float32)
    wt = jnp.repeat(w, L)
    b16 = jnp.broadcast_to(ffn_b.astype(jnp.float32), (L,))

    sc_matvec = pl.kernel(
        _scmv_body,
        out_type=jax.ShapeDtypeStruct((SCN,), jnp.float32),
        mesh=plsc.VectorSubcoreMesh(**_SC_MESH),
        compiler_params=pltpu.CompilerParams(needs_layout_passes=False),
        scratch_types=[
            pltpu.VMEM((LATENT_DIM, CW), jnp.float32),
            pltpu.VMEM((LATENT_DIM, CW), jnp.float32),
            pltpu.VMEM((LATENT_DIM * L,), jnp.float32),
            pltpu.VMEM((L,), jnp.float32),
            pltpu.VMEM((CW,), jnp.float32),
            pltpu.VMEM((CW,), jnp.float32),
            pltpu.SemaphoreType.DMA,
            pltpu.SemaphoreType.DMA,
            pltpu.SemaphoreType.DMA,
            pltpu.SemaphoreType.DMA,
        ],
    )
    y_sc = sc_matvec(et, wt, b16)

    y2 = pl.pallas_call(
        _matvec_body,
        grid=(NBLK_TC,),
        in_specs=[
            pl.BlockSpec((1, LATENT_DIM), lambda i: (0, 0)),
            pl.BlockSpec((LATENT_DIM, BLKW), lambda i: (0, i + IB0)),
            pl.BlockSpec((1, 1), lambda i: (0, 0), memory_space=pltpu.SMEM),
        ],
        out_specs=pl.BlockSpec((1, BLKW), lambda i: (0, i + IB0)),
        out_shape=jax.ShapeDtypeStruct((1, NUM_ITEMS), jnp.float32),
        compiler_params=pltpu.CompilerParams(
            dimension_semantics=("parallel",)),
    )(ffn_w, et, b2)
    y_tc = y2.reshape(NUM_ITEMS)
    y = lax.concatenate([y_sc, lax.slice(y_tc, (SCN,), (NUM_ITEMS,))], 0)

    gather = pl.kernel(
        _gather_body,
        out_type=jax.ShapeDtypeStruct((BATCH,), jnp.float32),
        mesh=plsc.VectorSubcoreMesh(**_SC_MESH),
        compiler_params=pltpu.CompilerParams(needs_layout_passes=False),
        scratch_types=[
            pltpu.VMEM((NCHUNK, CHUNK), jnp.int32),
            pltpu.VMEM((BPW,), jnp.float32),
            pltpu.SemaphoreType.DMA,
        ],
    )
    out = gather(y, idx)
    return out.reshape(BATCH, 1)


# speedup vs baseline: 1.2114x; 1.0515x over previous
import functools

import jax
import jax.numpy as jnp
from jax import lax
from jax.experimental import pallas as pl
from jax.experimental.pallas import tpu as pltpu
from jax.experimental.pallas import tpu_sc as plsc

NUM_ITEMS = 1000000
LATENT_DIM = 64
BATCH = 16384

NC = 2
NS = 16
L = 16
NW = NC * NS
BPW = BATCH // NW
CHUNK = 128
NCHUNK = BPW // CHUNK

BLKW = 32768
SCN = 622592
IB0 = SCN // BLKW
NBLK_TC = (NUM_ITEMS - SCN + BLKW - 1) // BLKW

WPW = SCN // NW
CW = 512
NCH = WPW // CW
GR = CW // L


def _matvec_body(w_ref, et_ref, b_ref, y_ref):
    y_ref[...] = jnp.dot(w_ref[...], et_ref[...],
                         preferred_element_type=jnp.float32) + b_ref[0, 0]


def _scmv_body(et_hbm, wt_hbm, b16_hbm, ysc_hbm,
               buf0, buf1, wt_v, b_v, yw0, yw1, sem0, sem1, osem0, osem1):
    wid = lax.axis_index("s") * NC + lax.axis_index("c")
    base = wid * WPW
    pltpu.sync_copy(wt_hbm, wt_v)
    pltpu.sync_copy(b16_hbm, b_v)
    pltpu.async_copy(et_hbm.at[:, pl.ds(base, CW)], buf0, sem0)

    def compute(buf, yw):
        def fbody(f, accs):
            wf = wt_v[pl.ds(f * L, L)]
            return tuple(a + buf[f, pl.ds(g * L, L)] * wf
                         for g, a in enumerate(accs))
        accs = lax.fori_loop(0, LATENT_DIM, fbody,
                             tuple(b_v[...] for _ in range(GR)))
        for g in range(GR):
            yw[pl.ds(g * L, L)] = accs[g]

    def pair(k, carry):
        c0 = 2 * k
        pltpu.make_async_copy(et_hbm.at[:, pl.ds(0, CW)], buf0, sem0).wait()
        pltpu.async_copy(
            et_hbm.at[:, pl.ds(base + (c0 + 1) * CW, CW)], buf1, sem1)

        @pl.when(k > 0)
        def _():
            pltpu.make_async_copy(
                yw0, ysc_hbm.at[pl.ds(0, CW)], osem0).wait()

        compute(buf0, yw0)
        pltpu.async_copy(yw0, ysc_hbm.at[pl.ds(base + c0 * CW, CW)], osem0)
        pltpu.make_async_copy(et_hbm.at[:, pl.ds(0, CW)], buf1, sem1).wait()

        @pl.when(k + 1 < NCH // 2)
        def _():
            pltpu.async_copy(
                et_hbm.at[:, pl.ds(base + (c0 + 2) * CW, CW)], buf0, sem0)

        @pl.when(k > 0)
        def _():
            pltpu.make_async_copy(
                yw1, ysc_hbm.at[pl.ds(0, CW)], osem1).wait()

        compute(buf1, yw1)
        pltpu.async_copy(
            yw1, ysc_hbm.at[pl.ds(base + (c0 + 1) * CW, CW)], osem1)
        return carry

    lax.fori_loop(0, NCH // 2, pair, 0)
    pltpu.make_async_copy(yw0, ysc_hbm.at[pl.ds(0, CW)], osem0).wait()
    pltpu.make_async_copy(yw1, ysc_hbm.at[pl.ds(0, CW)], osem1).wait()


def _gather_body(y_hbm, idx_hbm, out_hbm, idx_v, out_v, sem):
    wid = lax.axis_index("s") * NC + lax.axis_index("c")
    base = wid * BPW
    for c in range(NCHUNK):
        pltpu.sync_copy(idx_hbm.at[pl.ds(base + c * CHUNK, CHUNK)],
                        idx_v.at[c])
    copies = []
    for c in range(NCHUNK):
        copies.append(pltpu.async_copy(
            y_hbm.at[idx_v.at[c]],
            out_v.at[pl.ds(c * CHUNK, CHUNK)], sem))
    for cp in copies:
        cp.wait()
    pltpu.sync_copy(out_v, out_hbm.at[pl.ds(base, BPW)])


_SC_MESH = dict(core_axis_name="c", subcore_axis_name="s",
                num_cores=NC, num_subcores=NS)


@jax.jit
def kernel(item_indices, item_emb, ffn_w, ffn_b):
    idx = item_indices.astype(jnp.int32)
    et = jnp.swapaxes(item_emb, 0, 1)
    b2 = ffn_b.reshape(1, 1)
    w = ffn_w.reshape(LATENT_DIM).astype(jnp.float32)
    wt = jnp.repeat(w, L)
    b16 = jnp.broadcast_to(ffn_b.astype(jnp.float32), (L,))

    sc_matvec = pl.kernel(
        _scmv_body,
        out_type=jax.ShapeDtypeStruct((SCN,), jnp.float32),
        mesh=plsc.VectorSubcoreMesh(**_SC_MESH),
        compiler_params=pltpu.CompilerParams(needs_layout_passes=False),
        scratch_types=[
            pltpu.VMEM((LATENT_DIM, CW), jnp.float32),
            pltpu.VMEM((LATENT_DIM, CW), jnp.float32),
            pltpu.VMEM((LATENT_DIM * L,), jnp.float32),
            pltpu.VMEM((L,), jnp.float32),
            pltpu.VMEM((CW,), jnp.float32),
            pltpu.VMEM((CW,), jnp.float32),
            pltpu.SemaphoreType.DMA,
            pltpu.SemaphoreType.DMA,
            pltpu.SemaphoreType.DMA,
            pltpu.SemaphoreType.DMA,
        ],
    )
    y_sc = sc_matvec(et, wt, b16)

    y2 = pl.pallas_call(
        _matvec_body,
        grid=(NBLK_TC,),
        in_specs=[
            pl.BlockSpec((1, LATENT_DIM), lambda i: (0, 0)),
            pl.BlockSpec((LATENT_DIM, BLKW), lambda i: (0, i + IB0)),
            pl.BlockSpec((1, 1), lambda i: (0, 0), memory_space=pltpu.SMEM),
        ],
        out_specs=pl.BlockSpec((1, BLKW), lambda i: (0, i + IB0)),
        out_shape=jax.ShapeDtypeStruct((1, NUM_ITEMS), jnp.float32),
        compiler_params=pltpu.CompilerParams(
            dimension_semantics=("parallel",)),
    )(ffn_w, et, b2)
    y_tc = y2.reshape(NUM_ITEMS)
    y = lax.concatenate([y_sc, lax.slice(y_tc, (SCN,), (NUM_ITEMS,))], 0)

    gather = pl.kernel(
        _gather_body,
        out_type=jax.ShapeDtypeStruct((BATCH,), jnp.float32),
        mesh=plsc.VectorSubcoreMesh(**_SC_MESH),
        compiler_params=pltpu.CompilerParams(needs_layout_passes=False),
        scratch_types=[
            pltpu.VMEM((NCHUNK, CHUNK), jnp.int32),
            pltpu.VMEM((BPW,), jnp.float32),
            pltpu.SemaphoreType.DMA,
        ],
    )
    out = gather(y, idx)
    return out.reshape(BATCH, 1)
